# baseline (speedup 1.0000x reference)
import functools

import jax
import jax.numpy as jnp
from jax import lax
from jax.experimental import pallas as pl
from jax.experimental.pallas import tpu as pltpu
from jax.experimental.pallas import tpu_sc as plsc

N = 10000
E = 320000
D_NODE = 128
D_STATE = 128
D_MSG = 256
N_GRAPHS = 256
N_PROP = 5
D_GRAPH = 128

NS = 16
EPT = E // NS
K = 40
NCHT = EPT // K
SCH = 50
NSUP = NCHT // SCH
PAIRS = SCH // 2
KC = 80
NCHUNK_C = EPT // KC
RPT = 624
RTAIL = N - NS * RPT
RB = 104

RN = 1000
RE = 8000

_f32 = jnp.float32


def _enc_nodes_body(nf_ref, wn_ref, bn_ref, w1f_ref, w1t_ref,
                    h_ref, hf0_ref, hf1_ref, ht0_ref, ht1_ref):
    h = jnp.dot(nf_ref[...], wn_ref[...], preferred_element_type=_f32)
    h = h + bn_ref[...]
    h_ref[...] = h
    hf = jnp.dot(h, w1f_ref[...], preferred_element_type=_f32)
    ht = jnp.dot(h, w1t_ref[...], preferred_element_type=_f32)
    hf0_ref[...] = hf[:, :128]
    hf1_ref[...] = hf[:, 128:]
    ht0_ref[...] = ht[:, :128]
    ht1_ref[...] = ht[:, 128:]


def _enc_edges_body(ef_ref, w_ref, b_ref, ee0_ref, ee1_ref):
    ee = jnp.dot(ef_ref[...], w_ref[...], preferred_element_type=_f32)
    ee = ee + b_ref[...]
    ee0_ref[...] = ee[:, :128]
    ee1_ref[...] = ee[:, 128:]


def _layer_body(h_ref, s0_ref, s1_ref, cnt_ref, a_ref, b0_ref, b1_ref,
                b2u_ref, bupd_ref, w1f_ref, w1t_ref,
                hn_ref, hf0_ref, hf1_ref, ht0_ref, ht1_ref):
    h = h_ref[...]
    hn = jnp.dot(h, a_ref[...], preferred_element_type=_f32)
    s0 = s0_ref[...].astype(_f32)
    s1 = s1_ref[...].astype(_f32)
    hn = hn + jnp.dot(s0, b0_ref[...], preferred_element_type=_f32)
    hn = hn + jnp.dot(s1, b1_ref[...], preferred_element_type=_f32)
    hn = hn + cnt_ref[:, 0:1] * b2u_ref[...]
    hn = hn + bupd_ref[...]
    hn_ref[...] = hn
    hf = jnp.dot(hn, w1f_ref[...], preferred_element_type=_f32)
    ht = jnp.dot(hn, w1t_ref[...], preferred_element_type=_f32)
    hf0_ref[...] = hf[:, :128]
    hf1_ref[...] = hf[:, 128:]
    ht0_ref[...] = ht[:, :128]
    ht1_ref[...] = ht[:, 128:]


def _final_body(h_ref, gi_ref, wagg_ref, bagg_ref, wg_ref, bg_ref,
                out_ref, acc_ref):
    i = pl.program_id(0)

    @pl.when(i == 0)
    def _():
        acc_ref[...] = jnp.zeros_like(acc_ref)

    g = jnp.dot(h_ref[...], wagg_ref[...], preferred_element_type=_f32)
    g = g + bagg_ref[...]
    gated = jax.nn.sigmoid(g[:, :D_GRAPH]) * g[:, D_GRAPH:]
    gi = gi_ref[...].reshape(RN, 1)
    seg = lax.broadcasted_iota(jnp.int32, (1, N_GRAPHS), 1)
    onehot = (gi == seg).astype(_f32)
    acc_ref[...] += lax.dot_general(
        onehot, gated, (((0,), (0,)), ((), ())), preferred_element_type=_f32)

    @pl.when(i == (N // RN) - 1)
    def _():
        gv = jnp.dot(acc_ref[...], wg_ref[...], preferred_element_type=_f32)
        gv = gv + bg_ref[...]
        half = N_GRAPHS // 2
        re_ = lax.broadcasted_iota(jnp.int32, (half, N_GRAPHS), 0)
        ce_ = lax.broadcasted_iota(jnp.int32, (half, N_GRAPHS), 1)
        pe = (ce_ == 2 * re_).astype(_f32)
        po = (ce_ == 2 * re_ + 1).astype(_f32)
        d = jnp.dot(pe - po, gv, preferred_element_type=_f32)
        out_ref[...] = -jnp.sum(d * d, axis=1, keepdims=True)


_SC_MESH = plsc.VectorSubcoreMesh(core_axis_name="c", subcore_axis_name="s")


def _zero_rows(buf, nrows):
    def body(i, _):
        for j in range(8):
            buf[i, pl.ds(j * 16, 16)] = jnp.zeros((16,), _f32)
        return 0
    lax.fori_loop(0, nrows, body, 0, unroll=False)


@functools.partial(
    pl.kernel,
    out_type=[jax.ShapeDtypeStruct((N, 128), _f32),
              jax.ShapeDtypeStruct((N, 128), _f32)],
    mesh=_SC_MESH,
    scratch_types=[
        pltpu.VMEM((SCH, K), jnp.int32),
        pltpu.VMEM((SCH, K), jnp.int32),
        pltpu.VMEM((K, 128), _f32),
        pltpu.VMEM((K, 128), _f32),
        pltpu.VMEM((K, 128), _f32),
        pltpu.VMEM((K, 128), _f32),
        pltpu.VMEM((K, 128), _f32),
        pltpu.VMEM((K, 128), _f32),
        pltpu.VMEM_SHARED((N, 128), _f32),
        pltpu.SemaphoreType.DMA,
        pltpu.SemaphoreType.DMA,
    ],
)
def _edge_pass(hf0, hf1, ht0, ht1, ee0, ee1, fidx3, tidx3,
               s0_out, s1_out,
               idxf_v, idxt_v, av0, bv0, cv0, av1, bv1, cv1, s_sh,
               sem0, sem1):
    c = lax.axis_index("c")
    s = lax.axis_index("s")

    _zero_rows(av0, K)

    def zrow(r, _):
        pltpu.sync_copy(av0, s_sh.at[pl.ds(s * RPT + r * K, K)])
        return 0
    lax.fori_loop(0, RPT // K, zrow, 0, unroll=False)
    rleft = RPT - (RPT // K) * K
    pltpu.sync_copy(av0.at[pl.ds(0, rleft)],
                    s_sh.at[pl.ds(s * RPT + (RPT // K) * K, rleft)])

    @pl.when(s == NS - 1)
    def _():
        pltpu.sync_copy(av0.at[pl.ds(0, RTAIL)],
                        s_sh.at[pl.ds(NS * RPT, RTAIL)])
    plsc.subcore_barrier()

    def run(hf, ht, ee):
        for sc in range(NSUP):
            sb = s * NSUP + sc
            pltpu.sync_copy(fidx3.at[sb], idxf_v)
            pltpu.sync_copy(tidx3.at[sb], idxt_v)
            ebase = (s * NCHT + sc * SCH) * K

            def start(j, avs, bvs, cvs, sem):
                pltpu.async_copy(hf.at[idxf_v.at[j]], avs, sem)
                pltpu.async_copy(ht.at[idxt_v.at[j]], bvs, sem)
                pltpu.async_copy(ee.at[pl.ds(ebase + j * K, K)], cvs, sem)

            def finish(j, avs, bvs, cvs, sem):
                pltpu.make_async_copy(hf.at[idxf_v.at[j]], avs, sem).wait()
                pltpu.make_async_copy(ht.at[idxt_v.at[j]], bvs, sem).wait()
                pltpu.make_async_copy(
                    ee.at[pl.ds(ebase + j * K, K)], cvs, sem).wait()

                def rowfn(i, _):
                    for jj in range(8):
                        sl = pl.ds(jj * 16, 16)
                        avs[i, sl] = jnp.maximum(
                            avs[i, sl] + bvs[i, sl] + cvs[i, sl], 0.0)
                    return 0
                lax.fori_loop(0, K, rowfn, 0, unroll=False)
                pltpu.sync_copy(avs, s_sh.at[idxt_v.at[j]], add=True)

            start(0, av0, bv0, cv0, sem0)

            def pair(p, _):
                j0 = 2 * p
                start(j0 + 1, av1, bv1, cv1, sem1)
                finish(j0, av0, bv0, cv0, sem0)

                @pl.when(p < PAIRS - 1)
                def _():
                    start(j0 + 2, av0, bv0, cv0, sem0)

                finish(j0 + 1, av1, bv1, cv1, sem1)
                return 0
            lax.fori_loop(0, PAIRS, pair, 0, unroll=False)

    @pl.when(c == 0)
    def _():
        run(hf0, ht0, ee0)

    @pl.when(c == 1)
    def _():
        run(hf1, ht1, ee1)

    plsc.subcore_barrier()

    def writeout(out):
        pltpu.sync_copy(s_sh.at[pl.ds(s * RPT, RPT)],
                        out.at[pl.ds(s * RPT, RPT)])

        @pl.when(s == NS - 1)
        def _():
            pltpu.sync_copy(s_sh.at[pl.ds(NS * RPT, RTAIL)],
                            out.at[pl.ds(NS * RPT, RTAIL)])

    @pl.when(c == 0)
    def _():
        writeout(s0_out)

    @pl.when(c == 1)
    def _():
        writeout(s1_out)


@functools.partial(
    pl.kernel,
    out_type=jax.ShapeDtypeStruct((N, 16), _f32),
    mesh=_SC_MESH,
    scratch_types=[
        pltpu.VMEM((KC,), jnp.int32),
        pltpu.VMEM((KC, 16), _f32),
        pltpu.VMEM((RB, 16), _f32),
        pltpu.VMEM_SHARED((N, 16), _f32),
    ],
)
def _count_pass(tidx, cnt_out, idxt_v, ones_v, zb, c_sh):
    c = lax.axis_index("c")
    s = lax.axis_index("s")

    @pl.when(c == 0)
    def _():
        def zeros16(buf, nrows):
            def body(i, _):
                buf[i, :] = jnp.zeros((16,), _f32)
                return 0
            lax.fori_loop(0, nrows, body, 0, unroll=False)

        zeros16(zb, RB)
        for r in range(RPT // RB):
            pltpu.sync_copy(zb, c_sh.at[pl.ds(s * RPT + r * RB, RB)])

        @pl.when(s == NS - 1)
        def _():
            pltpu.sync_copy(zb.at[pl.ds(0, RTAIL)],
                            c_sh.at[pl.ds(NS * RPT, RTAIL)])

        def ones16(i, _):
            ones_v[i, :] = jnp.ones((16,), _f32)
            return 0
        lax.fori_loop(0, KC, ones16, 0, unroll=False)
        plsc.subcore_barrier()

        def chunk(k, _):
            base = s * EPT + k * KC
            pltpu.sync_copy(tidx.at[pl.ds(base, KC)], idxt_v)
            pltpu.sync_copy(ones_v, c_sh.at[idxt_v], add=True)
            return 0
        lax.fori_loop(0, NCHUNK_C, chunk, 0, unroll=False)
        plsc.subcore_barrier()
        pltpu.sync_copy(c_sh.at[pl.ds(s * RPT, RPT)],
                        cnt_out.at[pl.ds(s * RPT, RPT)])

        @pl.when(s == NS - 1)
        def _():
            pltpu.sync_copy(c_sh.at[pl.ds(NS * RPT, RTAIL)],
                            cnt_out.at[pl.ds(NS * RPT, RTAIL)])


def kernel(node_features, edge_features, from_idx, to_idx, graph_idx,
           W_enc_n, b_enc_n, W_enc_e, b_enc_e,
           W_msg1, b_msg1, W_msg2, b_msg2,
           W_upd, b_upd, W_agg, b_agg, W_graph, b_graph):
    w1f = W_msg1[:D_STATE]
    w1t = W_msg1[D_STATE:2 * D_STATE]
    w1e = W_msg1[2 * D_STATE:]
    a_upd = jnp.eye(D_STATE, dtype=_f32) + W_upd[:D_STATE]
    b_mat = jnp.dot(W_msg2, W_upd[D_STATE:])
    b2u = jnp.dot(b_msg2, W_upd[D_STATE:])
    w_ee = jnp.dot(W_enc_e, w1e)
    b_ee = jnp.dot(b_enc_e, w1e) + b_msg1

    row2 = lambda v: v.reshape(1, -1)

    cnt16 = _count_pass(to_idx)

    ge = E // RE
    ee0, ee1 = pl.pallas_call(
        _enc_edges_body,
        grid=(ge,),
        in_specs=[
            pl.BlockSpec((RE, 16), lambda i: (i, 0)),
            pl.BlockSpec((16, 256), lambda i: (0, 0)),
            pl.BlockSpec((1, 256), lambda i: (0, 0)),
        ],
        out_specs=[
            pl.BlockSpec((RE, 128), lambda i: (i, 0)),
            pl.BlockSpec((RE, 128), lambda i: (i, 0)),
        ],
        out_shape=[jax.ShapeDtypeStruct((E, 128), _f32),
                   jax.ShapeDtypeStruct((E, 128), _f32)],
    )(edge_features, w_ee, row2(b_ee))

    gn = N // RN
    wspec = lambda shape: pl.BlockSpec(shape, lambda i: (0, 0))
    h, hf0, hf1, ht0, ht1 = pl.pallas_call(
        _enc_nodes_body,
        grid=(gn,),
        in_specs=[
            pl.BlockSpec((RN, D_NODE), lambda i: (i, 0)),
            wspec((D_NODE, D_STATE)),
            wspec((1, D_STATE)),
            wspec((D_STATE, D_MSG)),
            wspec((D_STATE, D_MSG)),
        ],
        out_specs=[pl.BlockSpec((RN, 128), lambda i: (i, 0))] * 5,
        out_shape=[jax.ShapeDtypeStruct((N, 128), _f32)] * 5,
    )(node_features, W_enc_n, row2(b_enc_n), w1f, w1t)

    fidx3 = from_idx.reshape(E // (SCH * K), SCH, K)
    tidx3 = to_idx.reshape(E // (SCH * K), SCH, K)

    layer_call = pl.pallas_call(
        _layer_body,
        grid=(gn,),
        in_specs=[
            pl.BlockSpec((RN, 128), lambda i: (i, 0)),
            pl.BlockSpec((RN, 128), lambda i: (i, 0)),
            pl.BlockSpec((RN, 128), lambda i: (i, 0)),
            pl.BlockSpec((RN, 16), lambda i: (i, 0)),
            wspec((D_STATE, D_STATE)),
            wspec((128, 128)),
            wspec((128, 128)),
            wspec((1, 128)),
            wspec((1, 128)),
            wspec((D_STATE, D_MSG)),
            wspec((D_STATE, D_MSG)),
        ],
        out_specs=[pl.BlockSpec((RN, 128), lambda i: (i, 0))] * 5,
        out_shape=[jax.ShapeDtypeStruct((N, 128), _f32)] * 5,
    )

    for _ in range(N_PROP):
        s0, s1 = _edge_pass(hf0, hf1, ht0, ht1, ee0, ee1, fidx3, tidx3)
        h, hf0, hf1, ht0, ht1 = layer_call(
            h, s0, s1, cnt16, a_upd, b_mat[:128], b_mat[128:],
            row2(b2u), row2(b_upd), w1f, w1t)

    gi3 = graph_idx.reshape(gn, 1, RN)
    scores = pl.pallas_call(
        _final_body,
        grid=(gn,),
        in_specs=[
            pl.BlockSpec((RN, 128), lambda i: (i, 0)),
            pl.BlockSpec((1, 1, RN), lambda i: (i, 0, 0)),
            wspec((D_STATE, 2 * D_GRAPH)),
            wspec((1, 2 * D_GRAPH)),
            wspec((D_GRAPH, D_GRAPH)),
            wspec((1, D_GRAPH)),
        ],
        out_specs=pl.BlockSpec((N_GRAPHS // 2, 1), lambda i: (0, 0)),
        out_shape=jax.ShapeDtypeStruct((N_GRAPHS // 2, 1), _f32),
        scratch_shapes=[pltpu.VMEM((N_GRAPHS, D_GRAPH), _f32)],
    )(h, gi3, W_agg, row2(b_agg), W_graph, row2(b_graph))

    return scores[:, 0]


# speedup vs baseline: 1.0433x; 1.0433x over previous
import functools

import jax
import jax.numpy as jnp
from jax import lax
from jax.experimental import pallas as pl
from jax.experimental.pallas import tpu as pltpu
from jax.experimental.pallas import tpu_sc as plsc

N = 10000
E = 320000
D_NODE = 128
D_STATE = 128
D_MSG = 256
N_GRAPHS = 256
N_PROP = 5
D_GRAPH = 128

NS = 16
EPT = E // NS
K = 40
NCHT = EPT // K
SCH = 50
NSUP = NCHT // SCH
PAIRS = SCH // 2
KC = 80
NCHUNK_C = EPT // KC
RPT = 624
RTAIL = N - NS * RPT
RB = 104

RN = 1000
RE = 8000

_f32 = jnp.float32


def _enc_nodes_body(nf_ref, wn_ref, bn_ref, w1f_ref, w1t_ref,
                    h_ref, hf0_ref, hf1_ref, ht0_ref, ht1_ref):
    h = jnp.dot(nf_ref[...], wn_ref[...], preferred_element_type=_f32)
    h = h + bn_ref[...]
    h_ref[...] = h
    hf = jnp.dot(h, w1f_ref[...], preferred_element_type=_f32)
    ht = jnp.dot(h, w1t_ref[...], preferred_element_type=_f32)
    hf0_ref[...] = hf[:, :128]
    hf1_ref[...] = hf[:, 128:]
    ht0_ref[...] = ht[:, :128]
    ht1_ref[...] = ht[:, 128:]


def _enc_edges_body(ef_ref, w_ref, b_ref, ee0_ref, ee1_ref):
    ee = jnp.dot(ef_ref[...], w_ref[...], preferred_element_type=_f32)
    ee = ee + b_ref[...]
    ee0_ref[...] = ee[:, :128]
    ee1_ref[...] = ee[:, 128:]


def _layer_body(h_ref, s0_ref, s1_ref, cnt_ref, cntb_ref, a_ref, b0_ref,
                b1_ref, b2u_ref, bupd_ref, w1f_ref, w1t_ref,
                hn_ref, hf0_ref, hf1_ref, ht0_ref, ht1_ref):
    h = h_ref[...]
    hn = jnp.dot(h, a_ref[...], preferred_element_type=_f32)
    s0 = s0_ref[...].astype(_f32)
    s1 = s1_ref[...].astype(_f32)
    hn = hn + jnp.dot(s0, b0_ref[...], preferred_element_type=_f32)
    hn = hn + jnp.dot(s1, b1_ref[...], preferred_element_type=_f32)
    hn = hn + (cnt_ref[:, 0:1] + cntb_ref[:, 0:1]) * b2u_ref[...]
    hn = hn + bupd_ref[...]
    hn_ref[...] = hn
    hf = jnp.dot(hn, w1f_ref[...], preferred_element_type=_f32)
    ht = jnp.dot(hn, w1t_ref[...], preferred_element_type=_f32)
    hf0_ref[...] = hf[:, :128]
    hf1_ref[...] = hf[:, 128:]
    ht0_ref[...] = ht[:, :128]
    ht1_ref[...] = ht[:, 128:]


def _final_body(h_ref, gi_ref, wagg_ref, bagg_ref, wg_ref, bg_ref,
                out_ref, acc_ref):
    i = pl.program_id(0)

    @pl.when(i == 0)
    def _():
        acc_ref[...] = jnp.zeros_like(acc_ref)

    g = jnp.dot(h_ref[...], wagg_ref[...], preferred_element_type=_f32)
    g = g + bagg_ref[...]
    gated = jax.nn.sigmoid(g[:, :D_GRAPH]) * g[:, D_GRAPH:]
    gi = gi_ref[...].reshape(RN, 1)
    seg = lax.broadcasted_iota(jnp.int32, (1, N_GRAPHS), 1)
    onehot = (gi == seg).astype(_f32)
    acc_ref[...] += lax.dot_general(
        onehot, gated, (((0,), (0,)), ((), ())), preferred_element_type=_f32)

    @pl.when(i == (N // RN) - 1)
    def _():
        gv = jnp.dot(acc_ref[...], wg_ref[...], preferred_element_type=_f32)
        gv = gv + bg_ref[...]
        half = N_GRAPHS // 2
        re_ = lax.broadcasted_iota(jnp.int32, (half, N_GRAPHS), 0)
        ce_ = lax.broadcasted_iota(jnp.int32, (half, N_GRAPHS), 1)
        pe = (ce_ == 2 * re_).astype(_f32)
        po = (ce_ == 2 * re_ + 1).astype(_f32)
        d = jnp.dot(pe - po, gv, preferred_element_type=_f32)
        out_ref[...] = -jnp.sum(d * d, axis=1, keepdims=True)


_SC_MESH = plsc.VectorSubcoreMesh(core_axis_name="c", subcore_axis_name="s")


def _zero_rows(buf, nrows):
    def body(i, _):
        for j in range(8):
            buf[i, pl.ds(j * 16, 16)] = jnp.zeros((16,), _f32)
        return 0
    lax.fori_loop(0, nrows, body, 0, unroll=False)


@functools.partial(
    pl.kernel,
    out_type=[jax.ShapeDtypeStruct((N, 128), _f32),
              jax.ShapeDtypeStruct((N, 128), _f32)],
    mesh=_SC_MESH,
    scratch_types=[
        pltpu.VMEM((SCH, K), jnp.int32),
        pltpu.VMEM((SCH, K), jnp.int32),
        pltpu.VMEM((K, 128), _f32),
        pltpu.VMEM((K, 128), _f32),
        pltpu.VMEM((K, 128), _f32),
        pltpu.VMEM((K, 128), _f32),
        pltpu.VMEM((K, 128), _f32),
        pltpu.VMEM((K, 128), _f32),
        pltpu.VMEM_SHARED((N, 128), _f32),
        pltpu.SemaphoreType.DMA,
        pltpu.SemaphoreType.DMA,
    ],
)
def _edge_pass(hf0, hf1, ht0, ht1, ee0, ee1, fidx3, tidx3,
               s0_out, s1_out,
               idxf_v, idxt_v, av0, bv0, cv0, av1, bv1, cv1, s_sh,
               sem0, sem1):
    c = lax.axis_index("c")
    s = lax.axis_index("s")

    _zero_rows(av0, K)

    def zrow(r, _):
        pltpu.sync_copy(av0, s_sh.at[pl.ds(s * RPT + r * K, K)])
        return 0
    lax.fori_loop(0, RPT // K, zrow, 0, unroll=False)
    rleft = RPT - (RPT // K) * K
    pltpu.sync_copy(av0.at[pl.ds(0, rleft)],
                    s_sh.at[pl.ds(s * RPT + (RPT // K) * K, rleft)])

    @pl.when(s == NS - 1)
    def _():
        pltpu.sync_copy(av0.at[pl.ds(0, RTAIL)],
                        s_sh.at[pl.ds(NS * RPT, RTAIL)])
    plsc.subcore_barrier()

    def run(hf, ht, ee):
        for sc in range(NSUP):
            sb = s * NSUP + sc
            pltpu.sync_copy(fidx3.at[sb], idxf_v)
            pltpu.sync_copy(tidx3.at[sb], idxt_v)
            ebase = (s * NCHT + sc * SCH) * K

            def start(j, avs, bvs, cvs, sem):
                pltpu.async_copy(hf.at[idxf_v.at[j]], avs, sem)
                pltpu.async_copy(ht.at[idxt_v.at[j]], bvs, sem)
                pltpu.async_copy(ee.at[pl.ds(ebase + j * K, K)], cvs, sem)

            def finish(j, avs, bvs, cvs, sem):
                pltpu.make_async_copy(hf.at[idxf_v.at[j]], avs, sem).wait()
                pltpu.make_async_copy(ht.at[idxt_v.at[j]], bvs, sem).wait()
                pltpu.make_async_copy(
                    ee.at[pl.ds(ebase + j * K, K)], cvs, sem).wait()

                def rowfn(i, _):
                    for jj in range(8):
                        sl = pl.ds(jj * 16, 16)
                        avs[i, sl] = jnp.maximum(
                            avs[i, sl] + bvs[i, sl] + cvs[i, sl], 0.0)
                    return 0
                lax.fori_loop(0, K, rowfn, 0, unroll=False)
                pltpu.sync_copy(avs, s_sh.at[idxt_v.at[j]], add=True)

            start(0, av0, bv0, cv0, sem0)

            def pair(p, _):
                j0 = 2 * p
                start(j0 + 1, av1, bv1, cv1, sem1)
                finish(j0, av0, bv0, cv0, sem0)

                @pl.when(p < PAIRS - 1)
                def _():
                    start(j0 + 2, av0, bv0, cv0, sem0)

                finish(j0 + 1, av1, bv1, cv1, sem1)
                return 0
            lax.fori_loop(0, PAIRS, pair, 0, unroll=False)

    @pl.when(c == 0)
    def _():
        run(hf0, ht0, ee0)

    @pl.when(c == 1)
    def _():
        run(hf1, ht1, ee1)

    plsc.subcore_barrier()

    def writeout(out):
        pltpu.sync_copy(s_sh.at[pl.ds(s * RPT, RPT)],
                        out.at[pl.ds(s * RPT, RPT)])

        @pl.when(s == NS - 1)
        def _():
            pltpu.sync_copy(s_sh.at[pl.ds(NS * RPT, RTAIL)],
                            out.at[pl.ds(NS * RPT, RTAIL)])

    @pl.when(c == 0)
    def _():
        writeout(s0_out)

    @pl.when(c == 1)
    def _():
        writeout(s1_out)


@functools.partial(
    pl.kernel,
    out_type=jax.ShapeDtypeStruct((2, N, 16), _f32),
    mesh=_SC_MESH,
    scratch_types=[
        pltpu.VMEM((KC,), jnp.int32),
        pltpu.VMEM((KC, 16), _f32),
        pltpu.VMEM((RB, 16), _f32),
        pltpu.VMEM_SHARED((N, 16), _f32),
    ],
)
def _count_pass(tidx, cnt_out, idxt_v, ones_v, zb, c_sh):
    c = lax.axis_index("c")
    s = lax.axis_index("s")

    def zeros16(buf, nrows):
        def body(i, _):
            buf[i, :] = jnp.zeros((16,), _f32)
            return 0
        lax.fori_loop(0, nrows, body, 0, unroll=False)

    zeros16(zb, RB)
    for r in range(RPT // RB):
        pltpu.sync_copy(zb, c_sh.at[pl.ds(s * RPT + r * RB, RB)])

    @pl.when(s == NS - 1)
    def _():
        pltpu.sync_copy(zb.at[pl.ds(0, RTAIL)],
                        c_sh.at[pl.ds(NS * RPT, RTAIL)])

    def ones16(i, _):
        ones_v[i, :] = jnp.ones((16,), _f32)
        return 0
    lax.fori_loop(0, KC, ones16, 0, unroll=False)
    plsc.subcore_barrier()

    def chunk(k, _):
        base = c * (E // 2) + s * (EPT // 2) + k * KC
        pltpu.sync_copy(tidx.at[pl.ds(base, KC)], idxt_v)
        pltpu.sync_copy(ones_v, c_sh.at[idxt_v], add=True)
        return 0
    lax.fori_loop(0, NCHUNK_C // 2, chunk, 0, unroll=False)
    plsc.subcore_barrier()
    pltpu.sync_copy(c_sh.at[pl.ds(s * RPT, RPT)],
                    cnt_out.at[c].at[pl.ds(s * RPT, RPT)])

    @pl.when(s == NS - 1)
    def _():
        pltpu.sync_copy(c_sh.at[pl.ds(NS * RPT, RTAIL)],
                        cnt_out.at[c].at[pl.ds(NS * RPT, RTAIL)])


def kernel(node_features, edge_features, from_idx, to_idx, graph_idx,
           W_enc_n, b_enc_n, W_enc_e, b_enc_e,
           W_msg1, b_msg1, W_msg2, b_msg2,
           W_upd, b_upd, W_agg, b_agg, W_graph, b_graph):
    w1f = W_msg1[:D_STATE]
    w1t = W_msg1[D_STATE:2 * D_STATE]
    w1e = W_msg1[2 * D_STATE:]
    a_upd = jnp.eye(D_STATE, dtype=_f32) + W_upd[:D_STATE]
    b_mat = jnp.dot(W_msg2, W_upd[D_STATE:])
    b2u = jnp.dot(b_msg2, W_upd[D_STATE:])
    w_ee = jnp.dot(W_enc_e, w1e)
    b_ee = jnp.dot(b_enc_e, w1e) + b_msg1

    row2 = lambda v: v.reshape(1, -1)

    cnt2 = _count_pass(to_idx)
    cnt16a, cnt16b = cnt2[0], cnt2[1]

    ge = E // RE
    ee0, ee1 = pl.pallas_call(
        _enc_edges_body,
        grid=(ge,),
        in_specs=[
            pl.BlockSpec((RE, 16), lambda i: (i, 0)),
            pl.BlockSpec((16, 256), lambda i: (0, 0)),
            pl.BlockSpec((1, 256), lambda i: (0, 0)),
        ],
        out_specs=[
            pl.BlockSpec((RE, 128), lambda i: (i, 0)),
            pl.BlockSpec((RE, 128), lambda i: (i, 0)),
        ],
        out_shape=[jax.ShapeDtypeStruct((E, 128), _f32),
                   jax.ShapeDtypeStruct((E, 128), _f32)],
    )(edge_features, w_ee, row2(b_ee))

    gn = N // RN
    wspec = lambda shape: pl.BlockSpec(shape, lambda i: (0, 0))
    h, hf0, hf1, ht0, ht1 = pl.pallas_call(
        _enc_nodes_body,
        grid=(gn,),
        in_specs=[
            pl.BlockSpec((RN, D_NODE), lambda i: (i, 0)),
            wspec((D_NODE, D_STATE)),
            wspec((1, D_STATE)),
            wspec((D_STATE, D_MSG)),
            wspec((D_STATE, D_MSG)),
        ],
        out_specs=[pl.BlockSpec((RN, 128), lambda i: (i, 0))] * 5,
        out_shape=[jax.ShapeDtypeStruct((N, 128), _f32)] * 5,
    )(node_features, W_enc_n, row2(b_enc_n), w1f, w1t)

    fidx3 = from_idx.reshape(E // (SCH * K), SCH, K)
    tidx3 = to_idx.reshape(E // (SCH * K), SCH, K)

    layer_call = pl.pallas_call(
        _layer_body,
        grid=(gn,),
        in_specs=[
            pl.BlockSpec((RN, 128), lambda i: (i, 0)),
            pl.BlockSpec((RN, 128), lambda i: (i, 0)),
            pl.BlockSpec((RN, 128), lambda i: (i, 0)),
            pl.BlockSpec((RN, 16), lambda i: (i, 0)),
            pl.BlockSpec((RN, 16), lambda i: (i, 0)),
            wspec((D_STATE, D_STATE)),
            wspec((128, 128)),
            wspec((128, 128)),
            wspec((1, 128)),
            wspec((1, 128)),
            wspec((D_STATE, D_MSG)),
            wspec((D_STATE, D_MSG)),
        ],
        out_specs=[pl.BlockSpec((RN, 128), lambda i: (i, 0))] * 5,
        out_shape=[jax.ShapeDtypeStruct((N, 128), _f32)] * 5,
    )

    for _ in range(N_PROP):
        s0, s1 = _edge_pass(hf0, hf1, ht0, ht1, ee0, ee1, fidx3, tidx3)
        h, hf0, hf1, ht0, ht1 = layer_call(
            h, s0, s1, cnt16a, cnt16b, a_upd, b_mat[:128], b_mat[128:],
            row2(b2u), row2(b_upd), w1f, w1t)

    gi3 = graph_idx.reshape(gn, 1, RN)
    scores = pl.pallas_call(
        _final_body,
        grid=(gn,),
        in_specs=[
            pl.BlockSpec((RN, 128), lambda i: (i, 0)),
            pl.BlockSpec((1, 1, RN), lambda i: (i, 0, 0)),
            wspec((D_STATE, 2 * D_GRAPH)),
            wspec((1, 2 * D_GRAPH)),
            wspec((D_GRAPH, D_GRAPH)),
            wspec((1, D_GRAPH)),
        ],
        out_specs=pl.BlockSpec((N_GRAPHS // 2, 1), lambda i: (0, 0)),
        out_shape=jax.ShapeDtypeStruct((N_GRAPHS // 2, 1), _f32),
        scratch_shapes=[pltpu.VMEM((N_GRAPHS, D_GRAPH), _f32)],
    )(h, gi3, W_agg, row2(b_agg), W_graph, row2(b_graph))

    return scores[:, 0]
